# baseline (speedup 1.0000x reference)
import functools

import jax
import jax.numpy as jnp
from jax import lax
from jax.experimental import pallas as pl
from jax.experimental.pallas import tpu as pltpu
from jax.experimental.pallas import tpu_sc as plsc

_SCH = 32
_SK = 3
_SL = 1


def kernel(x):
    M, N = x.shape
    info = plsc.get_sparse_core_info()
    NC, NS = info.num_cores, info.num_subcores
    NW = NC * NS
    rpw = M // NW
    nch = rpw // _SCH

    mesh = plsc.VectorSubcoreMesh(core_axis_name="c", subcore_axis_name="s")

    @functools.partial(
        pl.kernel,
        out_type=jax.ShapeDtypeStruct((M, N), x.dtype),
        mesh=mesh,
        scratch_types=[
            pltpu.VMEM((_SK, _SCH, N), x.dtype),
            [pltpu.SemaphoreType.DMA] * _SK,
            [pltpu.SemaphoreType.DMA] * _SK,
        ],
    )
    def copy_k(x_hbm, out_hbm, bufs, in_sems, out_sems):
        wid = lax.axis_index("s") * NC + lax.axis_index("c")
        base = wid * rpw

        def in_copy(i):
            return pltpu.make_async_copy(
                x_hbm.at[pl.ds(base + i * _SCH, _SCH), :],
                bufs.at[i % _SK],
                in_sems[i % _SK],
            )

        def out_copy(i):
            return pltpu.make_async_copy(
                bufs.at[i % _SK],
                out_hbm.at[pl.ds(base + i * _SCH, _SCH), :],
                out_sems[i % _SK],
            )

        for i in range(-_SL, nch):
            if i >= 0:
                in_copy(i).wait()
                out_copy(i).start()
            j = i + _SL
            if 0 <= j < nch:
                if j >= _SK:
                    out_copy(j - _SK).wait()
                in_copy(j).start()
        for i in range(max(0, nch - _SK), nch):
            out_copy(i).wait()

    gathered = copy_k(x)
    sizes = jnp.asarray([M], dtype=jnp.int32)
    return (gathered, sizes)


# speedup vs baseline: 1.0071x; 1.0071x over previous
import functools

import jax
import jax.numpy as jnp
from jax import lax
from jax.experimental import pallas as pl
from jax.experimental.pallas import tpu as pltpu
from jax.experimental.pallas import tpu_sc as plsc

_SCH = 16
_SK = 7
_SL = 2


def kernel(x):
    M, N = x.shape
    info = plsc.get_sparse_core_info()
    NC, NS = info.num_cores, info.num_subcores
    NW = NC * NS
    rpw = M // NW
    nch = rpw // _SCH

    mesh = plsc.VectorSubcoreMesh(core_axis_name="c", subcore_axis_name="s")

    @functools.partial(
        pl.kernel,
        out_type=jax.ShapeDtypeStruct((M, N), x.dtype),
        mesh=mesh,
        scratch_types=[
            pltpu.VMEM((_SK, _SCH, N), x.dtype),
            [pltpu.SemaphoreType.DMA] * _SK,
            [pltpu.SemaphoreType.DMA] * _SK,
        ],
    )
    def copy_k(x_hbm, out_hbm, bufs, in_sems, out_sems):
        wid = lax.axis_index("s") * NC + lax.axis_index("c")
        base = wid * rpw

        def in_copy(i):
            return pltpu.make_async_copy(
                x_hbm.at[pl.ds(base + i * _SCH, _SCH), :],
                bufs.at[i % _SK],
                in_sems[i % _SK],
            )

        def out_copy(i):
            return pltpu.make_async_copy(
                bufs.at[i % _SK],
                out_hbm.at[pl.ds(base + i * _SCH, _SCH), :],
                out_sems[i % _SK],
            )

        for i in range(-_SL, nch):
            if i >= 0:
                in_copy(i).wait()
                out_copy(i).start()
            j = i + _SL
            if 0 <= j < nch:
                if j >= _SK:
                    out_copy(j - _SK).wait()
                in_copy(j).start()
        for i in range(max(0, nch - _SK), nch):
            out_copy(i).wait()

    gathered = copy_k(x)
    sizes = jnp.asarray([M], dtype=jnp.int32)
    return (gathered, sizes)
